# baseline (speedup 1.0000x reference)
import functools

import jax
import jax.numpy as jnp
from jax import lax
from jax.experimental import pallas as pl
from jax.experimental.pallas import tpu as pltpu
from jax.experimental.pallas import tpu_sc as plsc

N_NODES = 10000
N_EDGES = 320000
D = 128

NCORES = 2
NSUB = 16
NWORK = NCORES * NSUB
LANES = 16
EPW = N_EDGES // NWORK
CHUNK = 80
NCHUNK = EPW // CHUNK
GROUPS = CHUNK // LANES
EUNROLL = 4


def _tc_transform_body(circ_ref, w_ref, wc_ref, out_ref):
    w0 = w_ref[0]
    w1 = w_ref[1]
    m0 = w0 * wc_ref[0, 0] + w1 * wc_ref[1, 0]
    m1 = w0 * wc_ref[0, 1] + w1 * wc_ref[1, 1]
    m = jnp.concatenate([m0, m1], axis=1)
    out_ref[...] = jnp.dot(circ_ref[...], m,
                           preferred_element_type=jnp.float32)


def _tc_transform(circ, weight, wc):
    return pl.pallas_call(
        _tc_transform_body,
        out_shape=jax.ShapeDtypeStruct((N_NODES, 2 * D), jnp.float32),
        in_specs=[
            pl.BlockSpec(memory_space=pltpu.VMEM),
            pl.BlockSpec(memory_space=pltpu.VMEM),
            pl.BlockSpec(memory_space=pltpu.SMEM),
        ],
        out_specs=pl.BlockSpec(memory_space=pltpu.VMEM),
    )(circ, weight, wc)


def _sc_edge_body(t_hbm, dis_hbm, ci_hbm, di_hbm, o0_hbm, o1_hbm,
                  cidx_v, didx_v, rows_t, rows_d, o0_v, o1_v, sem_t, sem_d):
    wid = lax.axis_index("s") * NCORES + lax.axis_index("c")
    base = wid * EPW

    pltpu.sync_copy(ci_hbm.at[pl.ds(base, EPW)], cidx_v)
    pltpu.sync_copy(di_hbm.at[pl.ds(base, EPW)], didx_v)

    zero16 = jnp.zeros((LANES,), jnp.float32)

    @plsc.parallel_loop(0, EPW, step=LANES)
    def _(i):
        o0_v[pl.ds(i, LANES)] = zero16
        o1_v[pl.ds(i, LANES)] = zero16

    def chunk_body(c, carry):
        off = pl.multiple_of(c * CHUNK, CHUNK)
        cp_t = pltpu.make_async_copy(
            t_hbm.at[cidx_v.at[pl.ds(off, CHUNK)]], rows_t, sem_t)
        cp_d = pltpu.make_async_copy(
            dis_hbm.at[didx_v.at[pl.ds(off, CHUNK)]], rows_d, sem_d)
        cp_t.start()
        cp_d.start()
        cp_t.wait()
        cp_d.wait()

        @plsc.parallel_loop(0, CHUNK, step=1, unroll=EUNROLL)
        def _(e):
            a0 = jnp.zeros((LANES,), jnp.float32)
            a1 = jnp.zeros((LANES,), jnp.float32)
            for k in range(D // LANES):
                dv = rows_d[e, pl.ds(k * LANES, LANES)]
                t0 = rows_t[e, pl.ds(k * LANES, LANES)]
                t1 = rows_t[e, pl.ds(D + k * LANES, LANES)]
                a0 = a0 + t0 * dv
                a1 = a1 + t1 * dv
            ids = jnp.full((LANES,), off + e, jnp.int32)
            plsc.addupdate_scatter(o0_v, [ids], a0)
            plsc.addupdate_scatter(o1_v, [ids], a1)

        return carry

    lax.fori_loop(0, NCHUNK, chunk_body, 0)

    @plsc.parallel_loop(0, EPW, step=LANES)
    def _(i):
        o0_v[pl.ds(i, LANES)] = jnp.maximum(o0_v[pl.ds(i, LANES)], 0.0)
        o1_v[pl.ds(i, LANES)] = jnp.maximum(o1_v[pl.ds(i, LANES)], 0.0)

    pltpu.sync_copy(o0_v, o0_hbm.at[pl.ds(base, EPW)])
    pltpu.sync_copy(o1_v, o1_hbm.at[pl.ds(base, EPW)])


@functools.lru_cache(maxsize=1)
def _sc_edge():
  return pl.kernel(
    _sc_edge_body,
    out_type=(
        jax.ShapeDtypeStruct((N_EDGES,), jnp.float32),
        jax.ShapeDtypeStruct((N_EDGES,), jnp.float32),
    ),
    mesh=plsc.VectorSubcoreMesh(core_axis_name="c", subcore_axis_name="s",
                                num_cores=NCORES, num_subcores=NSUB),
    compiler_params=pltpu.CompilerParams(needs_layout_passes=False),
    scratch_types=[
        pltpu.VMEM((EPW,), jnp.int32),
        pltpu.VMEM((EPW,), jnp.int32),
        pltpu.VMEM((CHUNK, 2 * D), jnp.float32),
        pltpu.VMEM((CHUNK, D), jnp.float32),
        pltpu.VMEM((EPW,), jnp.float32),
        pltpu.VMEM((EPW,), jnp.float32),
        pltpu.SemaphoreType.DMA,
        pltpu.SemaphoreType.DMA,
    ],
  )


@jax.jit
def kernel(circ_inputs, dis_inputs, weight, weight_classifier,
           circ_indices, dis_indices):
    t = _tc_transform(circ_inputs, weight, weight_classifier)
    o0, o1 = _sc_edge()(t, dis_inputs,
                      circ_indices.astype(jnp.int32),
                      dis_indices.astype(jnp.int32))
    return jnp.stack([o0, o1], axis=1)


# speedup vs baseline: 1.7706x; 1.7706x over previous
import functools

import jax
import jax.numpy as jnp
from jax import lax
from jax.experimental import pallas as pl
from jax.experimental.pallas import tpu as pltpu
from jax.experimental.pallas import tpu_sc as plsc

N_NODES = 10000
N_EDGES = 320000
D = 128

NCORES = 2
NSUB = 16
NWORK = NCORES * NSUB
LANES = 16
EPW = N_EDGES // NWORK
CHUNK = 80
NCHUNK = EPW // CHUNK
GROUPS = CHUNK // LANES
EUNROLL = 4


def _tc_transform_body(circ_ref, w_ref, wc_ref, out_ref):
    w0 = w_ref[0]
    w1 = w_ref[1]
    m0 = w0 * wc_ref[0, 0] + w1 * wc_ref[1, 0]
    m1 = w0 * wc_ref[0, 1] + w1 * wc_ref[1, 1]
    m = jnp.concatenate([m0, m1], axis=1)
    out_ref[...] = jnp.dot(circ_ref[...], m,
                           preferred_element_type=jnp.float32)


def _tc_transform(circ, weight, wc):
    return pl.pallas_call(
        _tc_transform_body,
        out_shape=jax.ShapeDtypeStruct((N_NODES, 2 * D), jnp.float32),
        in_specs=[
            pl.BlockSpec(memory_space=pltpu.VMEM),
            pl.BlockSpec(memory_space=pltpu.VMEM),
            pl.BlockSpec(memory_space=pltpu.SMEM),
        ],
        out_specs=pl.BlockSpec(memory_space=pltpu.VMEM),
    )(circ, weight, wc)


def _sc_edge_body(t_hbm, dis_hbm, ci_hbm, di_hbm, o0_hbm, o1_hbm,
                  cidx_v, didx_v, rows_t, rows_d, o0_v, o1_v, sem_t, sem_d):
    wid = lax.axis_index("s") * NCORES + lax.axis_index("c")
    base = wid * EPW

    pltpu.sync_copy(ci_hbm.at[pl.ds(base, EPW)], cidx_v)
    pltpu.sync_copy(di_hbm.at[pl.ds(base, EPW)], didx_v)

    def chunk_body(c, carry):
        off = pl.multiple_of(c * CHUNK, CHUNK)
        cp_t = pltpu.make_async_copy(
            t_hbm.at[cidx_v.at[pl.ds(off, CHUNK)]], rows_t, sem_t)
        cp_d = pltpu.make_async_copy(
            dis_hbm.at[didx_v.at[pl.ds(off, CHUNK)]], rows_d, sem_d)
        cp_t.start()
        cp_d.start()
        cp_t.wait()
        cp_d.wait()

        lane15 = lax.iota(jnp.int32, LANES) == (LANES - 1)

        @plsc.parallel_loop(0, CHUNK, step=1, unroll=EUNROLL)
        def _(e):
            a0 = jnp.zeros((LANES,), jnp.float32)
            a1 = jnp.zeros((LANES,), jnp.float32)
            for k in range(D // LANES):
                dv = rows_d[e, pl.ds(k * LANES, LANES)]
                t0 = rows_t[e, pl.ds(k * LANES, LANES)]
                t1 = rows_t[e, pl.ds(D + k * LANES, LANES)]
                a0 = a0 + t0 * dv
                a1 = a1 + t1 * dv
            c0 = jnp.maximum(plsc.cumsum(a0), 0.0)
            c1 = jnp.maximum(plsc.cumsum(a1), 0.0)
            ids = jnp.full((LANES,), off + e, jnp.int32)
            plsc.store_scatter(o0_v, [ids], c0, mask=lane15)
            plsc.store_scatter(o1_v, [ids], c1, mask=lane15)

        return carry

    lax.fori_loop(0, NCHUNK, chunk_body, 0)

    pltpu.sync_copy(o0_v, o0_hbm.at[pl.ds(base, EPW)])
    pltpu.sync_copy(o1_v, o1_hbm.at[pl.ds(base, EPW)])


@functools.lru_cache(maxsize=1)
def _sc_edge():
  return pl.kernel(
    _sc_edge_body,
    out_type=(
        jax.ShapeDtypeStruct((N_EDGES,), jnp.float32),
        jax.ShapeDtypeStruct((N_EDGES,), jnp.float32),
    ),
    mesh=plsc.VectorSubcoreMesh(core_axis_name="c", subcore_axis_name="s",
                                num_cores=NCORES, num_subcores=NSUB),
    compiler_params=pltpu.CompilerParams(needs_layout_passes=False),
    scratch_types=[
        pltpu.VMEM((EPW,), jnp.int32),
        pltpu.VMEM((EPW,), jnp.int32),
        pltpu.VMEM((CHUNK, 2 * D), jnp.float32),
        pltpu.VMEM((CHUNK, D), jnp.float32),
        pltpu.VMEM((EPW,), jnp.float32),
        pltpu.VMEM((EPW,), jnp.float32),
        pltpu.SemaphoreType.DMA,
        pltpu.SemaphoreType.DMA,
    ],
  )


@jax.jit
def kernel(circ_inputs, dis_inputs, weight, weight_classifier,
           circ_indices, dis_indices):
    t = _tc_transform(circ_inputs, weight, weight_classifier)
    o0, o1 = _sc_edge()(t, dis_inputs,
                      circ_indices.astype(jnp.int32),
                      dis_indices.astype(jnp.int32))
    return jnp.stack([o0, o1], axis=1)


# speedup vs baseline: 2.5701x; 1.4515x over previous
import functools

import jax
import jax.numpy as jnp
from jax import lax
from jax.experimental import pallas as pl
from jax.experimental.pallas import tpu as pltpu
from jax.experimental.pallas import tpu_sc as plsc

N_NODES = 10000
N_EDGES = 320000
D = 128

NCORES = 2
NSUB = 16
NWORK = NCORES * NSUB
LANES = 16
EPW = N_EDGES // NWORK
CHUNK = 80
NCHUNK = EPW // CHUNK
GROUPS = CHUNK // LANES
EUNROLL = 4


def _tc_transform_body(circ_ref, w_ref, wc_ref, out_ref):
    w0 = w_ref[0]
    w1 = w_ref[1]
    m0 = w0 * wc_ref[0, 0] + w1 * wc_ref[1, 0]
    m1 = w0 * wc_ref[0, 1] + w1 * wc_ref[1, 1]
    m = jnp.concatenate([m0, m1], axis=1)
    out_ref[...] = jnp.dot(circ_ref[...], m,
                           preferred_element_type=jnp.float32)


def _tc_transform(circ, weight, wc):
    return pl.pallas_call(
        _tc_transform_body,
        out_shape=jax.ShapeDtypeStruct((N_NODES, 2 * D), jnp.float32),
        in_specs=[
            pl.BlockSpec(memory_space=pltpu.VMEM),
            pl.BlockSpec(memory_space=pltpu.VMEM),
            pl.BlockSpec(memory_space=pltpu.SMEM),
        ],
        out_specs=pl.BlockSpec(memory_space=pltpu.VMEM),
    )(circ, weight, wc)


def _sc_edge_body(t_hbm, dis_hbm, ci_hbm, di_hbm, o0_hbm, o1_hbm,
                  cidx_v, didx_v, rows_t0, rows_d0, rows_t1, rows_d1,
                  o0_v, o1_v, sem0, sem1):
    wid = lax.axis_index("s") * NCORES + lax.axis_index("c")
    base = wid * EPW

    pltpu.sync_copy(ci_hbm.at[pl.ds(base, EPW)], cidx_v)
    pltpu.sync_copy(di_hbm.at[pl.ds(base, EPW)], didx_v)

    lane15 = lax.iota(jnp.int32, LANES) == (LANES - 1)

    def copies(c, rt, rd, sem):
        off = pl.multiple_of(c * CHUNK, CHUNK)
        return (
            pltpu.make_async_copy(
                t_hbm.at[cidx_v.at[pl.ds(off, CHUNK)]], rt, sem),
            pltpu.make_async_copy(
                dis_hbm.at[didx_v.at[pl.ds(off, CHUNK)]], rd, sem),
        )

    def gather_start(c, rt, rd, sem):
        cp_t, cp_d = copies(c, rt, rd, sem)
        cp_t.start()
        cp_d.start()

    def gather_wait(c, rt, rd, sem):
        cp_t, cp_d = copies(c, rt, rd, sem)
        cp_t.wait()
        cp_d.wait()

    def compute(c, rt, rd):
        off = pl.multiple_of(c * CHUNK, CHUNK)

        @plsc.parallel_loop(0, CHUNK, step=1, unroll=EUNROLL)
        def _(e):
            a0 = jnp.zeros((LANES,), jnp.float32)
            a1 = jnp.zeros((LANES,), jnp.float32)
            for k in range(D // LANES):
                dv = rd[e, pl.ds(k * LANES, LANES)]
                t0 = rt[e, pl.ds(k * LANES, LANES)]
                t1 = rt[e, pl.ds(D + k * LANES, LANES)]
                a0 = a0 + t0 * dv
                a1 = a1 + t1 * dv
            c0 = jnp.maximum(plsc.cumsum(a0), 0.0)
            c1 = jnp.maximum(plsc.cumsum(a1), 0.0)
            ids = jnp.full((LANES,), off + e, jnp.int32)
            plsc.store_scatter(o0_v, [ids], c0, mask=lane15)
            plsc.store_scatter(o1_v, [ids], c1, mask=lane15)

    gather_start(0, rows_t0, rows_d0, sem0)

    def pair_body(i, carry):
        c0 = i * 2
        c1 = c0 + 1
        gather_wait(c0, rows_t0, rows_d0, sem0)
        gather_start(c1, rows_t1, rows_d1, sem1)
        compute(c0, rows_t0, rows_d0)
        gather_wait(c1, rows_t1, rows_d1, sem1)
        gather_start(c1 + 1, rows_t0, rows_d0, sem0)
        compute(c1, rows_t1, rows_d1)
        return carry

    lax.fori_loop(0, NCHUNK // 2, pair_body, 0)
    gather_wait(NCHUNK - 1, rows_t0, rows_d0, sem0)
    compute(NCHUNK - 1, rows_t0, rows_d0)

    pltpu.sync_copy(o0_v, o0_hbm.at[pl.ds(base, EPW)])
    pltpu.sync_copy(o1_v, o1_hbm.at[pl.ds(base, EPW)])


@functools.lru_cache(maxsize=1)
def _sc_edge():
  return pl.kernel(
    _sc_edge_body,
    out_type=(
        jax.ShapeDtypeStruct((N_EDGES,), jnp.float32),
        jax.ShapeDtypeStruct((N_EDGES,), jnp.float32),
    ),
    mesh=plsc.VectorSubcoreMesh(core_axis_name="c", subcore_axis_name="s",
                                num_cores=NCORES, num_subcores=NSUB),
    compiler_params=pltpu.CompilerParams(needs_layout_passes=False),
    scratch_types=[
        pltpu.VMEM((EPW,), jnp.int32),
        pltpu.VMEM((EPW,), jnp.int32),
        pltpu.VMEM((CHUNK, 2 * D), jnp.float32),
        pltpu.VMEM((CHUNK, D), jnp.float32),
        pltpu.VMEM((CHUNK, 2 * D), jnp.float32),
        pltpu.VMEM((CHUNK, D), jnp.float32),
        pltpu.VMEM((EPW,), jnp.float32),
        pltpu.VMEM((EPW,), jnp.float32),
        pltpu.SemaphoreType.DMA,
        pltpu.SemaphoreType.DMA,
    ],
  )


@jax.jit
def kernel(circ_inputs, dis_inputs, weight, weight_classifier,
           circ_indices, dis_indices):
    t = _tc_transform(circ_inputs, weight, weight_classifier)
    o0, o1 = _sc_edge()(t, dis_inputs,
                      circ_indices.astype(jnp.int32),
                      dis_indices.astype(jnp.int32))
    return jnp.stack([o0, o1], axis=1)
